# baseline (device time: 83074 ns/iter reference)
import jax
import jax.numpy as jnp
from jax import lax
from jax.experimental import pallas as pl
from jax.experimental.pallas import tpu as pltpu

N_DEV = 4
BM = 512
EPS = 1e-5

_DEVID = getattr(pl, "DeviceIdType", None) or pltpu.DeviceIdType
_sem_signal = getattr(pl, "semaphore_signal", None) or pltpu.semaphore_signal
_sem_wait = getattr(pl, "semaphore_wait", None) or pltpu.semaphore_wait


def kernel(x, gamma, beta):
    m, n_loc = x.shape
    n_glob = float(N_DEV * n_loc)
    nblk = m // BM
    assert nblk % 2 == 0
    g2 = gamma.reshape(1, n_loc)
    b2 = beta.reshape(1, n_loc)

    def body(x_ref, g_ref, b_ref, out_ref,
             xin_ref, xc_ref, stage_ref, loc_ref, snd_ref, comm_ref, ms_ref,
             xin_sems, stage_sems, send_sems, recv_sems):
        my = lax.axis_index("i")

        def in_dma(i, slot):
            return pltpu.make_async_copy(
                x_ref.at[pl.ds(i * BM, BM), :],
                xin_ref.at[slot],
                xin_sems.at[slot],
            )

        def p0_compute(i, slot):
            xs = xin_ref[slot]
            rows = pl.ds(i * BM, BM)
            loc_ref[rows, 0:1] = jnp.sum(xs, axis=1, keepdims=True)
            loc_ref[rows, 1:2] = jnp.sum(xs * xs, axis=1, keepdims=True)
            xc_ref[rows, :] = xs.astype(jnp.bfloat16)

        in_dma(0, 0).start()

        def p0_body(j, carry):
            i = 2 * j
            in_dma(i, 0).wait()
            in_dma(i + 1, 1).start()
            p0_compute(i, 0)
            in_dma(i + 1, 1).wait()

            @pl.when(i + 2 < nblk)
            def _():
                in_dma(i + 2, 0).start()

            p0_compute(i + 1, 1)
            return carry

        lax.fori_loop(0, nblk // 2, p0_body, 0)

        snd_ref[...] = jnp.transpose(loc_ref[...], (1, 0))

        barrier = pltpu.get_barrier_semaphore()
        for k in range(1, N_DEV):
            _sem_signal(
                barrier, inc=1,
                device_id=((my + k) % N_DEV,),
                device_id_type=_DEVID.MESH,
            )
        _sem_wait(barrier, N_DEV - 1)

        sends = []
        for k in range(1, N_DEV):
            tgt = (my + k) % N_DEV
            rdma = pltpu.make_async_remote_copy(
                src_ref=snd_ref,
                dst_ref=comm_ref.at[my],
                send_sem=send_sems.at[k - 1],
                recv_sem=recv_sems.at[my],
                device_id=(tgt,),
                device_id_type=_DEVID.MESH,
            )
            rdma.start()
            sends.append(rdma)
        for rdma in sends:
            rdma.wait_send()

        tot = snd_ref[...]
        for k in range(1, N_DEV):
            src = (my + k) % N_DEV
            recv = pltpu.make_async_remote_copy(
                src_ref=snd_ref,
                dst_ref=comm_ref.at[src],
                send_sem=send_sems.at[0],
                recv_sem=recv_sems.at[src],
                device_id=(src,),
                device_id_type=_DEVID.MESH,
            )
            recv.wait_recv()
            tot = tot + comm_ref[src]

        mean = tot[0:1, :] / n_glob
        ex2 = tot[1:2, :] / n_glob
        rstd = lax.rsqrt(ex2 - mean * mean + EPS)
        ms_ref[...] = jnp.transpose(
            jnp.concatenate([mean, rstd], axis=0), (1, 0)
        )

        gb = g_ref[...].astype(jnp.bfloat16)
        bb = b_ref[...].astype(jnp.bfloat16)

        def out_dma(i, slot):
            return pltpu.make_async_copy(
                stage_ref.at[slot],
                out_ref.at[pl.ds(i * BM, BM), :],
                stage_sems.at[slot],
            )

        def p1_compute(i, slot):
            ri = pl.ds(i * BM, BM)
            xb = xc_ref[ri, :]
            mu = ms_ref[ri, 0:1].astype(jnp.bfloat16)
            rs = ms_ref[ri, 1:2].astype(jnp.bfloat16)
            stage_ref[slot] = gb * ((xb - mu) * rs) + bb

        def p1_body(j, carry):
            i = 2 * j

            @pl.when(i >= 2)
            def _():
                out_dma(i - 2, 0).wait()

            p1_compute(i, 0)
            out_dma(i, 0).start()

            @pl.when(i >= 2)
            def _():
                out_dma(i - 1, 1).wait()

            p1_compute(i + 1, 1)
            out_dma(i + 1, 1).start()
            return carry

        lax.fori_loop(0, nblk // 2, p1_body, 0)
        out_dma(nblk - 2, 0).wait()
        out_dma(nblk - 1, 1).wait()

    return pl.pallas_call(
        body,
        out_shape=jax.ShapeDtypeStruct((m, n_loc), jnp.bfloat16),
        in_specs=[
            pl.BlockSpec(memory_space=pl.ANY),
            pl.BlockSpec(memory_space=pltpu.VMEM),
            pl.BlockSpec(memory_space=pltpu.VMEM),
        ],
        out_specs=pl.BlockSpec(memory_space=pl.ANY),
        scratch_shapes=[
            pltpu.VMEM((2, BM, n_loc), jnp.float32),
            pltpu.VMEM((m, n_loc), jnp.bfloat16),
            pltpu.VMEM((2, BM, n_loc), jnp.bfloat16),
            pltpu.VMEM((m, 2), jnp.float32),
            pltpu.VMEM((2, m), jnp.float32),
            pltpu.VMEM((N_DEV, 2, m), jnp.float32),
            pltpu.VMEM((m, 2), jnp.float32),
            pltpu.SemaphoreType.DMA((2,)),
            pltpu.SemaphoreType.DMA((2,)),
            pltpu.SemaphoreType.DMA((N_DEV - 1,)),
            pltpu.SemaphoreType.DMA((N_DEV,)),
        ],
        compiler_params=pltpu.CompilerParams(
            collective_id=0,
            vmem_limit_bytes=60 * 1024 * 1024,
        ),
    )(x, g2, b2)
